# baseline (device time: 192562 ns/iter reference)
import jax
import jax.numpy as jnp
from jax import lax
from jax.experimental import pallas as pl
from jax.experimental.pallas import tpu as pltpu

T_BLK = 8
W_CORR = 128


def kernel(x, A, B, C):
    Bb, S, D = x.shape
    N = A.shape[1]

    def body(x_ref, a_ref, b_ref, c_ref, out_ref, comm_ref, send_sem, recv_sem):
        my_x = lax.axis_index("x")
        my_y = lax.axis_index("y")

        dAt = jnp.exp(a_ref[:, :]).T[None]

        def step_block(i, h):
            t0 = i * T_BLK
            xb = x_ref[:, pl.ds(t0, T_BLK), :]
            bb = b_ref[:, pl.ds(t0, T_BLK), :]
            cb = c_ref[:, pl.ds(t0, T_BLK), :]
            ys = []
            for k in range(T_BLK):
                h = h * dAt + xb[:, k, :][:, None, :] * bb[:, k, :][:, :, None]
                ys.append(jnp.sum(h * cb[:, k, :][:, :, None], axis=1))
            out_ref[:, pl.ds(t0, T_BLK), :] = jnp.stack(ys, axis=1)
            return h

        h0 = jnp.zeros((Bb, N, D), jnp.float32)
        h_end = lax.fori_loop(0, S // T_BLK, step_block, h0)

        @pl.when(my_x == 0)
        def _():
            comm_ref[...] = h_end
            send = pltpu.make_async_remote_copy(
                src_ref=comm_ref,
                dst_ref=comm_ref,
                send_sem=send_sem,
                recv_sem=recv_sem,
                device_id=(1, my_y),
                device_id_type=pl.DeviceIdType.MESH,
            )
            send.start()
            send.wait_send()

        @pl.when(my_x == 1)
        def _():
            recv = pltpu.make_async_remote_copy(
                src_ref=comm_ref,
                dst_ref=comm_ref,
                send_sem=send_sem,
                recv_sem=recv_sem,
                device_id=(0, my_y),
                device_id_type=pl.DeviceIdType.MESH,
            )
            recv.wait_recv()
            hc0 = comm_ref[...]

            def corr_block(i, hc):
                t0 = i * T_BLK
                cb = c_ref[:, pl.ds(t0, T_BLK), :]
                dys = []
                for k in range(T_BLK):
                    hc = hc * dAt
                    dys.append(jnp.sum(hc * cb[:, k, :][:, :, None], axis=1))
                out_ref[:, pl.ds(t0, T_BLK), :] = (
                    out_ref[:, pl.ds(t0, T_BLK), :] + jnp.stack(dys, axis=1)
                )
                return hc

            lax.fori_loop(0, W_CORR // T_BLK, corr_block, hc0)

    return pl.pallas_call(
        body,
        out_shape=jax.ShapeDtypeStruct((Bb, S, D), jnp.float32),
        in_specs=[pl.BlockSpec(memory_space=pltpu.VMEM)] * 4,
        out_specs=pl.BlockSpec(memory_space=pltpu.VMEM),
        scratch_shapes=[
            pltpu.VMEM((Bb, N, D), jnp.float32),
            pltpu.SemaphoreType.DMA,
            pltpu.SemaphoreType.DMA,
        ],
    )(x, A, B, C)


# device time: 131168 ns/iter; 1.4681x vs baseline; 1.4681x over previous
import jax
import jax.numpy as jnp
from jax import lax
from jax.experimental import pallas as pl
from jax.experimental.pallas import tpu as pltpu

T_BLK = 8
CH = 128
W_CORR = 128


def kernel(x, A, B, C):
    Bb, S, D = x.shape
    N = A.shape[1]
    Dh = D // 2
    BPC = CH // T_BLK
    NCH = S // CH

    def body(x_ref, a_ref, b_ref, c_ref, out_ref,
             comm_ref, hs_sem, hr_sem, ysend_sems, yrecv_sems):
        my_x = lax.axis_index("x")
        my_y = lax.axis_index("y")
        d0 = my_y * Dh

        dAt = jnp.exp(a_ref[pl.ds(d0, Dh), :]).T[None]

        def chunk_copy(c):
            sl = (slice(None), pl.ds(c * CH, CH), pl.ds(d0, Dh))
            return pltpu.make_async_remote_copy(
                src_ref=out_ref.at[sl],
                dst_ref=out_ref.at[sl],
                send_sem=ysend_sems.at[c],
                recv_sem=yrecv_sems.at[c],
                device_id=(my_x, 1 - my_y),
                device_id_type=pl.DeviceIdType.MESH,
            )

        def h_copy(target_x):
            return pltpu.make_async_remote_copy(
                src_ref=comm_ref,
                dst_ref=comm_ref,
                send_sem=hs_sem,
                recv_sem=hr_sem,
                device_id=(target_x, my_y),
                device_id_type=pl.DeviceIdType.MESH,
            )

        def step_block(i, h):
            t0 = i * T_BLK
            xb = x_ref[:, pl.ds(t0, T_BLK), pl.ds(d0, Dh)]
            bb = b_ref[:, pl.ds(t0, T_BLK), :]
            cb = c_ref[:, pl.ds(t0, T_BLK), :]
            ys = []
            for k in range(T_BLK):
                h = h * dAt + xb[:, k, :][:, None, :] * bb[:, k, :][:, :, None]
                ys.append(jnp.sum(h * cb[:, k, :][:, :, None], axis=1))
            out_ref[:, pl.ds(t0, T_BLK), pl.ds(d0, Dh)] = jnp.stack(ys, axis=1)

            c = i // BPC
            @pl.when(((i + 1) % BPC == 0) & ((my_x == 0) | (c > 0)))
            def _():
                chunk_copy(c).start()
            return h

        h0 = jnp.zeros((Bb, N, Dh), jnp.float32)
        h_end = lax.fori_loop(0, S // T_BLK, step_block, h0)

        @pl.when(my_x == 0)
        def _():
            comm_ref[...] = h_end
            send = h_copy(1)
            send.start()
            send.wait_send()

        @pl.when(my_x == 1)
        def _():
            h_copy(0).wait_recv()
            hc0 = comm_ref[...]

            def corr_block(i, hc):
                t0 = i * T_BLK
                cb = c_ref[:, pl.ds(t0, T_BLK), :]
                dys = []
                for k in range(T_BLK):
                    hc = hc * dAt
                    dys.append(jnp.sum(hc * cb[:, k, :][:, :, None], axis=1))
                sl = (slice(None), pl.ds(t0, T_BLK), pl.ds(d0, Dh))
                out_ref[sl] = out_ref[sl] + jnp.stack(dys, axis=1)
                return hc

            lax.fori_loop(0, W_CORR // T_BLK, corr_block, hc0)
            chunk_copy(0).start()

        for c in range(NCH):
            chunk_copy(c).wait_send()
            chunk_copy(c).wait_recv()

    return pl.pallas_call(
        body,
        out_shape=jax.ShapeDtypeStruct((Bb, S, D), jnp.float32),
        in_specs=[pl.BlockSpec(memory_space=pltpu.VMEM)] * 4,
        out_specs=pl.BlockSpec(memory_space=pltpu.VMEM),
        scratch_shapes=[
            pltpu.VMEM((Bb, N, Dh), jnp.float32),
            pltpu.SemaphoreType.DMA,
            pltpu.SemaphoreType.DMA,
            pltpu.SemaphoreType.DMA((NCH,)),
            pltpu.SemaphoreType.DMA((NCH,)),
        ],
    )(x, A, B, C)


# device time: 120605 ns/iter; 1.5966x vs baseline; 1.0876x over previous
import jax
import jax.numpy as jnp
from jax import lax
from jax.experimental import pallas as pl
from jax.experimental.pallas import tpu as pltpu

T_BLK = 16
CH = 64
W_CORR = 64


def kernel(x, A, B, C):
    Bb, S, D = x.shape
    N = A.shape[1]
    Dh = D // 2
    BPC = CH // T_BLK
    NCH = S // CH

    def body(x_ref, a_ref, b_ref, c_ref, out_ref,
             comm_ref, hs_sem, hr_sem, ysend_sems, yrecv_sems):
        my_x = lax.axis_index("x")
        my_y = lax.axis_index("y")
        d0 = my_y * Dh

        dAt = jnp.exp(a_ref[pl.ds(d0, Dh), :]).T[None]

        def chunk_copy(c):
            sl = (slice(None), pl.ds(c * CH, CH), pl.ds(d0, Dh))
            return pltpu.make_async_remote_copy(
                src_ref=out_ref.at[sl],
                dst_ref=out_ref.at[sl],
                send_sem=ysend_sems.at[c],
                recv_sem=yrecv_sems.at[c],
                device_id=(my_x, 1 - my_y),
                device_id_type=pl.DeviceIdType.MESH,
            )

        def h_copy(target_x):
            return pltpu.make_async_remote_copy(
                src_ref=comm_ref,
                dst_ref=comm_ref,
                send_sem=hs_sem,
                recv_sem=hr_sem,
                device_id=(target_x, my_y),
                device_id_type=pl.DeviceIdType.MESH,
            )

        def step_block(i, h):
            t0 = i * T_BLK
            xb = x_ref[:, pl.ds(t0, T_BLK), pl.ds(d0, Dh)]
            bb = b_ref[:, pl.ds(t0, T_BLK), :]
            cb = c_ref[:, pl.ds(t0, T_BLK), :]
            ys = []
            for k in range(T_BLK):
                h = h * dAt + xb[:, k, :][:, None, :] * bb[:, k, :][:, :, None]
                ys.append(jnp.sum(h * cb[:, k, :][:, :, None], axis=1))
            out_ref[:, pl.ds(t0, T_BLK), pl.ds(d0, Dh)] = jnp.stack(ys, axis=1)

            c = i // BPC
            @pl.when(((i + 1) % BPC == 0) & ((my_x == 0) | (c > 0)))
            def _():
                chunk_copy(c).start()
            return h

        h0 = jnp.zeros((Bb, N, Dh), jnp.float32)
        h_end = lax.fori_loop(0, S // T_BLK, step_block, h0)

        @pl.when(my_x == 0)
        def _():
            comm_ref[...] = h_end
            send = h_copy(1)
            send.start()
            send.wait_send()

        @pl.when(my_x == 1)
        def _():
            h_copy(0).wait_recv()
            hc0 = comm_ref[...]

            def corr_block(i, hc):
                t0 = i * T_BLK
                cb = c_ref[:, pl.ds(t0, T_BLK), :]
                dys = []
                for k in range(T_BLK):
                    hc = hc * dAt
                    dys.append(jnp.sum(hc * cb[:, k, :][:, :, None], axis=1))
                sl = (slice(None), pl.ds(t0, T_BLK), pl.ds(d0, Dh))
                out_ref[sl] = out_ref[sl] + jnp.stack(dys, axis=1)
                return hc

            lax.fori_loop(0, W_CORR // T_BLK, corr_block, hc0)
            chunk_copy(0).start()

        for c in range(NCH):
            chunk_copy(c).wait_send()
            chunk_copy(c).wait_recv()

    return pl.pallas_call(
        body,
        out_shape=jax.ShapeDtypeStruct((Bb, S, D), jnp.float32),
        in_specs=[pl.BlockSpec(memory_space=pltpu.VMEM)] * 4,
        out_specs=pl.BlockSpec(memory_space=pltpu.VMEM),
        scratch_shapes=[
            pltpu.VMEM((Bb, N, Dh), jnp.float32),
            pltpu.SemaphoreType.DMA,
            pltpu.SemaphoreType.DMA,
            pltpu.SemaphoreType.DMA((NCH,)),
            pltpu.SemaphoreType.DMA((NCH,)),
        ],
    )(x, A, B, C)


# device time: 116539 ns/iter; 1.6523x vs baseline; 1.0349x over previous
import jax
import jax.numpy as jnp
from jax import lax
from jax.experimental import pallas as pl
from jax.experimental.pallas import tpu as pltpu

T_BLK = 16
CH = 32
W_CORR = 32


def kernel(x, A, B, C):
    Bb, S, D = x.shape
    N = A.shape[1]
    Dh = D // 2
    BPC = CH // T_BLK
    NCH = S // CH

    def body(x_ref, a_ref, b_ref, c_ref, out_ref,
             comm_ref, hs_sem, hr_sem, ysend_sems, yrecv_sems):
        my_x = lax.axis_index("x")
        my_y = lax.axis_index("y")
        d0 = my_y * Dh

        dAt = jnp.exp(a_ref[pl.ds(d0, Dh), :]).T[None]

        def chunk_copy(c):
            sl = (slice(None), pl.ds(c * CH, CH), pl.ds(d0, Dh))
            return pltpu.make_async_remote_copy(
                src_ref=out_ref.at[sl],
                dst_ref=out_ref.at[sl],
                send_sem=ysend_sems.at[c],
                recv_sem=yrecv_sems.at[c],
                device_id=(my_x, 1 - my_y),
                device_id_type=pl.DeviceIdType.MESH,
            )

        def h_copy(target_x):
            return pltpu.make_async_remote_copy(
                src_ref=comm_ref,
                dst_ref=comm_ref,
                send_sem=hs_sem,
                recv_sem=hr_sem,
                device_id=(target_x, my_y),
                device_id_type=pl.DeviceIdType.MESH,
            )

        def step_block(i, h):
            t0 = i * T_BLK
            xb = x_ref[:, pl.ds(t0, T_BLK), pl.ds(d0, Dh)]
            bb = b_ref[:, pl.ds(t0, T_BLK), :]
            cb = c_ref[:, pl.ds(t0, T_BLK), :]
            ys = []
            for k in range(T_BLK):
                h = h * dAt + xb[:, k, :][:, None, :] * bb[:, k, :][:, :, None]
                ys.append(jnp.sum(h * cb[:, k, :][:, :, None], axis=1))
            out_ref[:, pl.ds(t0, T_BLK), pl.ds(d0, Dh)] = jnp.stack(ys, axis=1)

            c = i // BPC
            @pl.when(((i + 1) % BPC == 0) & ((my_x == 0) | (c > 0)))
            def _():
                chunk_copy(c).start()
            return h

        h0 = jnp.zeros((Bb, N, Dh), jnp.float32)
        h_end = lax.fori_loop(0, S // T_BLK, step_block, h0)

        @pl.when(my_x == 0)
        def _():
            comm_ref[...] = h_end
            send = h_copy(1)
            send.start()
            send.wait_send()

        @pl.when(my_x == 1)
        def _():
            h_copy(0).wait_recv()
            hc0 = comm_ref[...]

            def corr_block(i, hc):
                t0 = i * T_BLK
                cb = c_ref[:, pl.ds(t0, T_BLK), :]
                dys = []
                for k in range(T_BLK):
                    hc = hc * dAt
                    dys.append(jnp.sum(hc * cb[:, k, :][:, :, None], axis=1))
                sl = (slice(None), pl.ds(t0, T_BLK), pl.ds(d0, Dh))
                out_ref[sl] = out_ref[sl] + jnp.stack(dys, axis=1)
                return hc

            lax.fori_loop(0, W_CORR // T_BLK, corr_block, hc0)
            chunk_copy(0).start()

        for c in range(NCH):
            chunk_copy(c).wait_send()
            chunk_copy(c).wait_recv()

    return pl.pallas_call(
        body,
        out_shape=jax.ShapeDtypeStruct((Bb, S, D), jnp.float32),
        in_specs=[pl.BlockSpec(memory_space=pltpu.VMEM)] * 4,
        out_specs=pl.BlockSpec(memory_space=pltpu.VMEM),
        scratch_shapes=[
            pltpu.VMEM((Bb, N, Dh), jnp.float32),
            pltpu.SemaphoreType.DMA,
            pltpu.SemaphoreType.DMA,
            pltpu.SemaphoreType.DMA((NCH,)),
            pltpu.SemaphoreType.DMA((NCH,)),
        ],
    )(x, A, B, C)
